# baseline (speedup 1.0000x reference)
import jax
import jax.numpy as jnp
from jax.experimental import pallas as pl
from jax.experimental.pallas import tpu as pltpu

D_MODEL = 1024
D_FF = 4096
N_EXP = 8
SEQ = 2048
K_SEL = 2
R_TOT = SEQ * K_SEL
BM = 256
BN = 512
NT = R_TOT // BM + N_EXP
NN = D_FF // BN


def _router_kernel(x_ref, wg_ref, probs_ref, gates_ref, idx_ref):
    x = x_ref[...]
    logits = jax.lax.dot_general(
        x, wg_ref[...], (((1,), (1,)), ((), ())),
        preferred_element_type=jnp.float32)
    m = jnp.max(logits, axis=-1, keepdims=True)
    ex = jnp.exp(logits - m)
    probs = ex / jnp.sum(ex, axis=-1, keepdims=True)
    probs_ref[...] = probs
    idx = jax.lax.broadcasted_iota(jnp.int32, probs.shape, 1)
    m1 = jnp.max(probs, axis=-1, keepdims=True)
    i1 = jnp.min(jnp.where(probs == m1, idx, N_EXP), axis=-1, keepdims=True)
    p2 = jnp.where(idx == i1, -1.0, probs)
    m2 = jnp.max(p2, axis=-1, keepdims=True)
    i2 = jnp.min(jnp.where(p2 == m2, idx, N_EXP), axis=-1, keepdims=True)
    denom = m1 + m2
    gates_ref[...] = jnp.concatenate([m1 / denom, m2 / denom], axis=1)
    idx_ref[...] = jnp.concatenate([i1, i2], axis=1)


def _gather_kernel(tok_ref, x_ref, xs_ref):
    j = pl.program_id(0)

    def body(i, _):
        tok = tok_ref[j * BM + i]
        xs_ref[pl.ds(i, 1), :] = x_ref[pl.ds(tok, 1), :]
        return 0

    jax.lax.fori_loop(0, BM, body, 0)


def _ffn_kernel(e_ref, tok_ref, xs_ref, g_ref, we_ref, wv_ref, wo_ref,
                out_ref, rows_ref):
    n = pl.program_id(0)
    j = pl.program_id(1)

    @pl.when((n == 0) & (j == 0))
    def _init():
        out_ref[...] = jnp.zeros_like(out_ref)

    xs = xs_ref[...]
    g = jax.lax.dot_general(xs, we_ref[0], (((1,), (1,)), ((), ())),
                            preferred_element_type=jnp.float32)
    v = jax.lax.dot_general(xs, wv_ref[0], (((1,), (1,)), ((), ())),
                            preferred_element_type=jnp.float32)
    h = (0.5 * g * (1.0 + jax.lax.erf(g * 0.7071067811865476))) * v
    h = h * g_ref[0]
    part = jax.lax.dot_general(h, wo_ref[0], (((1,), (1,)), ((), ())),
                               preferred_element_type=jnp.float32)
    base = j * BM

    @pl.when(n == 0)
    def _first():
        rows_ref[pl.ds(base, BM), :] = part

    @pl.when(n > 0)
    def _rest():
        rows_ref[pl.ds(base, BM), :] += part

    @pl.when(n == NN - 1)
    def _scatter():
        def body(i, _):
            tok = tok_ref[base + i]
            out_ref[pl.ds(tok, 1), :] += rows_ref[pl.ds(base + i, 1), :]
            return 0

        jax.lax.fori_loop(0, BM, body, 0)


@jax.jit
def kernel(x, Wg, We, Wv, Wo):
    x2 = x.reshape(SEQ, D_MODEL)
    probs, gates2, idx2 = pl.pallas_call(
        _router_kernel,
        out_shape=(
            jax.ShapeDtypeStruct((SEQ, N_EXP), jnp.float32),
            jax.ShapeDtypeStruct((SEQ, K_SEL), jnp.float32),
            jax.ShapeDtypeStruct((SEQ, K_SEL), jnp.int32),
        ),
    )(x2, Wg)

    e_flat = idx2.reshape(-1)
    g_flat = gates2.reshape(-1)
    perm = jnp.argsort(e_flat)
    tok_sorted = (perm // K_SEL).astype(jnp.int32)
    g_sorted = g_flat[perm]
    counts = jnp.bincount(e_flat, length=N_EXP)
    csum = jnp.cumsum(counts)
    offsets = csum - counts
    ntiles = (counts + BM - 1) // BM
    tb_incl = jnp.cumsum(ntiles)
    tile_base = tb_incl - ntiles
    j = jnp.arange(NT)
    e_of_tile = jnp.searchsorted(tb_incl, j, side="right")
    valid_tile = e_of_tile < N_EXP
    e_clamped = jnp.where(valid_tile, e_of_tile, 0).astype(jnp.int32)
    k_in_e = j - tile_base[e_clamped]
    row_start = offsets[e_clamped] + k_in_e * BM
    s = jnp.arange(NT * BM)
    jj = s // BM
    r = row_start[jj] + (s % BM)
    valid = valid_tile[jj] & (r < csum[e_clamped[jj]])
    r_safe = jnp.clip(r, 0, R_TOT - 1)
    tok_padded = jnp.where(valid, tok_sorted[r_safe], 0).astype(jnp.int32)
    g_padded = jnp.where(valid, g_sorted[r_safe], 0.0).reshape(NT, BM, 1)

    xs = pl.pallas_call(
        _gather_kernel,
        grid=(NT,),
        in_specs=[
            pl.BlockSpec(memory_space=pltpu.SMEM),
            pl.BlockSpec((SEQ, D_MODEL), lambda jt: (0, 0)),
        ],
        out_specs=pl.BlockSpec((BM, D_MODEL), lambda jt: (jt, 0)),
        out_shape=jax.ShapeDtypeStruct((NT * BM, D_MODEL), jnp.float32),
    )(tok_padded, x2)

    out = pl.pallas_call(
        _ffn_kernel,
        grid_spec=pltpu.PrefetchScalarGridSpec(
            num_scalar_prefetch=2,
            grid=(NN, NT),
            in_specs=[
                pl.BlockSpec((BM, D_MODEL), lambda n, jt, e_t, tok: (jt, 0)),
                pl.BlockSpec((1, BM, 1), lambda n, jt, e_t, tok: (jt, 0, 0)),
                pl.BlockSpec((1, BN, D_MODEL),
                             lambda n, jt, e_t, tok: (e_t[jt], n, 0)),
                pl.BlockSpec((1, BN, D_MODEL),
                             lambda n, jt, e_t, tok: (e_t[jt], n, 0)),
                pl.BlockSpec((1, D_MODEL, BN),
                             lambda n, jt, e_t, tok: (e_t[jt], 0, n)),
            ],
            out_specs=pl.BlockSpec((SEQ, D_MODEL), lambda n, jt, e_t, tok: (0, 0)),
            scratch_shapes=[pltpu.VMEM((NT * BM, D_MODEL), jnp.float32)],
        ),
        out_shape=jax.ShapeDtypeStruct((SEQ, D_MODEL), jnp.float32),
    )(e_clamped, tok_padded, xs, g_padded, We, Wv, Wo)

    return out.reshape(1, SEQ, D_MODEL), probs.reshape(1, SEQ, N_EXP)


# speedup vs baseline: 1.1611x; 1.1611x over previous
import jax
import jax.numpy as jnp
from jax.experimental import pallas as pl
from jax.experimental.pallas import tpu as pltpu

D_MODEL = 1024
D_FF = 4096
N_EXP = 8
SEQ = 2048
K_SEL = 2
R_TOT = SEQ * K_SEL
BM = 256
BN = 512
NT = R_TOT // BM + N_EXP
NT_PAD = 32
NN = D_FF // BN


def _router_kernel(x_ref, wg_ref, probs_ref, gates_ref, slot_ref, etile_ref):
    x = x_ref[...]
    logits = jax.lax.dot_general(
        x, wg_ref[...], (((1,), (1,)), ((), ())),
        preferred_element_type=jnp.float32)
    m = jnp.max(logits, axis=-1, keepdims=True)
    ex = jnp.exp(logits - m)
    probs = ex / jnp.sum(ex, axis=-1, keepdims=True)
    probs_ref[...] = probs
    idx = jax.lax.broadcasted_iota(jnp.int32, probs.shape, 1)
    m1 = jnp.max(probs, axis=-1, keepdims=True)
    i1 = jnp.min(jnp.where(probs == m1, idx, N_EXP), axis=-1, keepdims=True)
    oh1 = (idx == i1).astype(jnp.float32)
    p2 = jnp.where(idx == i1, -1.0, probs)
    m2 = jnp.max(p2, axis=-1, keepdims=True)
    i2 = jnp.min(jnp.where(p2 == m2, idx, N_EXP), axis=-1, keepdims=True)
    oh2 = (idx == i2).astype(jnp.float32)
    denom = m1 + m2
    gates_ref[...] = jnp.concatenate([m1 / denom, m2 / denom], axis=1)

    both = (oh1 + oh2).astype(jnp.bfloat16)
    ia = jax.lax.broadcasted_iota(jnp.int32, (SEQ, SEQ), 0)
    ib = jax.lax.broadcasted_iota(jnp.int32, (SEQ, SEQ), 1)
    ltri = (ib < ia).astype(jnp.bfloat16)
    cum = jax.lax.dot_general(ltri, both, (((1,), (0,)), ((), ())),
                              preferred_element_type=jnp.float32)
    rank1 = jnp.sum(oh1 * cum, axis=-1, keepdims=True)
    rank2 = jnp.sum(oh2 * cum, axis=-1, keepdims=True)

    counts = jnp.sum(oh1 + oh2, axis=0, keepdims=True)
    ntiles = jnp.floor((counts + (BM - 1)) * (1.0 / BM))
    ea = jax.lax.broadcasted_iota(jnp.int32, (N_EXP, N_EXP), 0)
    eb = jax.lax.broadcasted_iota(jnp.int32, (N_EXP, N_EXP), 1)
    t8 = (ea < eb).astype(jnp.bfloat16)
    tile_base = jax.lax.dot_general(
        ntiles.astype(jnp.bfloat16), t8, (((1,), (0,)), ((), ())),
        preferred_element_type=jnp.float32)
    tb1 = jnp.sum(oh1 * tile_base, axis=-1, keepdims=True)
    tb2 = jnp.sum(oh2 * tile_base, axis=-1, keepdims=True)
    slot1 = (tb1 * BM + rank1).astype(jnp.int32)
    slot2 = (tb2 * BM + rank2).astype(jnp.int32)
    slot_ref[...] = jnp.concatenate([slot1, slot2], axis=1)

    tb_incl = tile_base + ntiles
    ji = jax.lax.broadcasted_iota(jnp.int32, (NT_PAD, N_EXP), 0).astype(jnp.float32)
    etile = jnp.sum((jnp.broadcast_to(tb_incl, (NT_PAD, N_EXP)) <= ji)
                    .astype(jnp.float32), axis=-1, keepdims=True)
    etile_ref[...] = jnp.minimum(etile, N_EXP - 1).astype(jnp.int32)


def _gather_kernel(tok_ref, x_ref, xs_ref):
    j = pl.program_id(0)

    def body(i, _):
        tok = tok_ref[j * BM + i]
        xs_ref[pl.ds(i, 1), :] = x_ref[pl.ds(tok, 1), :]
        return 0

    jax.lax.fori_loop(0, BM, body, 0)


def _ffn_kernel(e_ref, tok_ref, xs_ref, g_ref, we_ref, wv_ref, wo_ref,
                out_ref, rows_ref):
    n = pl.program_id(0)
    j = pl.program_id(1)

    @pl.when((n == 0) & (j == 0))
    def _init():
        out_ref[...] = jnp.zeros_like(out_ref)

    xs = xs_ref[...]
    g = jax.lax.dot_general(xs, we_ref[0], (((1,), (1,)), ((), ())),
                            preferred_element_type=jnp.float32)
    v = jax.lax.dot_general(xs, wv_ref[0], (((1,), (1,)), ((), ())),
                            preferred_element_type=jnp.float32)
    h = (0.5 * g * (1.0 + jax.lax.erf(g * 0.7071067811865476))) * v
    h = h * g_ref[0]
    part = jax.lax.dot_general(h, wo_ref[0], (((1,), (1,)), ((), ())),
                               preferred_element_type=jnp.float32)
    base = j * BM

    @pl.when(n == 0)
    def _first():
        rows_ref[pl.ds(base, BM), :] = part

    @pl.when(n > 0)
    def _rest():
        rows_ref[pl.ds(base, BM), :] += part

    @pl.when(n == NN - 1)
    def _scatter():
        def body(i, _):
            tok = tok_ref[base + i]
            out_ref[pl.ds(tok, 1), :] += rows_ref[pl.ds(base + i, 1), :]
            return 0

        jax.lax.fori_loop(0, BM, body, 0)


@jax.jit
def kernel(x, Wg, We, Wv, Wo):
    x2 = x.reshape(SEQ, D_MODEL)
    probs, gates2, slot2, etile = pl.pallas_call(
        _router_kernel,
        out_shape=(
            jax.ShapeDtypeStruct((SEQ, N_EXP), jnp.float32),
            jax.ShapeDtypeStruct((SEQ, K_SEL), jnp.float32),
            jax.ShapeDtypeStruct((SEQ, K_SEL), jnp.int32),
            jax.ShapeDtypeStruct((NT_PAD, 1), jnp.int32),
        ),
    )(x2, Wg)

    slot_flat = slot2.reshape(-1)
    tok_ids = jnp.arange(R_TOT, dtype=jnp.int32) // K_SEL
    tok_padded = jnp.zeros(NT * BM, jnp.int32).at[slot_flat].set(tok_ids)
    g_padded = (jnp.zeros(NT * BM, jnp.float32)
                .at[slot_flat].set(gates2.reshape(-1))
                .reshape(NT, BM, 1))
    e_of_tile = etile.reshape(NT_PAD)

    xs = pl.pallas_call(
        _gather_kernel,
        grid=(NT,),
        in_specs=[
            pl.BlockSpec(memory_space=pltpu.SMEM),
            pl.BlockSpec((SEQ, D_MODEL), lambda jt: (0, 0)),
        ],
        out_specs=pl.BlockSpec((BM, D_MODEL), lambda jt: (jt, 0)),
        out_shape=jax.ShapeDtypeStruct((NT * BM, D_MODEL), jnp.float32),
    )(tok_padded, x2)

    out = pl.pallas_call(
        _ffn_kernel,
        grid_spec=pltpu.PrefetchScalarGridSpec(
            num_scalar_prefetch=2,
            grid=(NN, NT),
            in_specs=[
                pl.BlockSpec((BM, D_MODEL), lambda n, jt, e_t, tok: (jt, 0)),
                pl.BlockSpec((1, BM, 1), lambda n, jt, e_t, tok: (jt, 0, 0)),
                pl.BlockSpec((1, BN, D_MODEL),
                             lambda n, jt, e_t, tok: (e_t[jt], n, 0)),
                pl.BlockSpec((1, BN, D_MODEL),
                             lambda n, jt, e_t, tok: (e_t[jt], n, 0)),
                pl.BlockSpec((1, D_MODEL, BN),
                             lambda n, jt, e_t, tok: (e_t[jt], 0, n)),
            ],
            out_specs=pl.BlockSpec((SEQ, D_MODEL), lambda n, jt, e_t, tok: (0, 0)),
            scratch_shapes=[pltpu.VMEM((NT * BM, D_MODEL), jnp.float32)],
        ),
        out_shape=jax.ShapeDtypeStruct((SEQ, D_MODEL), jnp.float32),
    )(e_of_tile, tok_padded, xs, g_padded, We, Wv, Wo)

    return out.reshape(1, SEQ, D_MODEL), probs.reshape(1, SEQ, N_EXP)


# speedup vs baseline: 1.2083x; 1.0406x over previous
import jax
import jax.numpy as jnp
from jax.experimental import pallas as pl
from jax.experimental.pallas import tpu as pltpu

D_MODEL = 1024
D_FF = 4096
N_EXP = 8
SEQ = 2048
K_SEL = 2
R_TOT = SEQ * K_SEL
BM = 256
BN = 512
NT = R_TOT // BM + N_EXP
NT_PAD = 32
NN = D_FF // BN


def _router_kernel(x_ref, wg_ref, probs_ref, gates_ref, slot_ref, etile_ref):
    x = x_ref[...]
    logits = jax.lax.dot_general(
        x, wg_ref[...], (((1,), (1,)), ((), ())),
        preferred_element_type=jnp.float32)
    m = jnp.max(logits, axis=-1, keepdims=True)
    ex = jnp.exp(logits - m)
    probs = ex / jnp.sum(ex, axis=-1, keepdims=True)
    probs_ref[...] = probs
    idx = jax.lax.broadcasted_iota(jnp.int32, probs.shape, 1)
    m1 = jnp.max(probs, axis=-1, keepdims=True)
    i1 = jnp.min(jnp.where(probs == m1, idx, N_EXP), axis=-1, keepdims=True)
    oh1 = (idx == i1).astype(jnp.float32)
    p2 = jnp.where(idx == i1, -1.0, probs)
    m2 = jnp.max(p2, axis=-1, keepdims=True)
    i2 = jnp.min(jnp.where(p2 == m2, idx, N_EXP), axis=-1, keepdims=True)
    oh2 = (idx == i2).astype(jnp.float32)
    denom = m1 + m2
    gates_ref[...] = jnp.concatenate([m1 / denom, m2 / denom], axis=1)

    both = (oh1 + oh2).astype(jnp.bfloat16)
    ia = jax.lax.broadcasted_iota(jnp.int32, (SEQ, SEQ), 0)
    ib = jax.lax.broadcasted_iota(jnp.int32, (SEQ, SEQ), 1)
    ltri = (ib < ia).astype(jnp.bfloat16)
    cum = jax.lax.dot_general(ltri, both, (((1,), (0,)), ((), ())),
                              preferred_element_type=jnp.float32)
    rank1 = jnp.sum(oh1 * cum, axis=-1, keepdims=True)
    rank2 = jnp.sum(oh2 * cum, axis=-1, keepdims=True)

    counts = jnp.sum(oh1 + oh2, axis=0, keepdims=True)
    ntiles = jnp.floor((counts + (BM - 1)) * (1.0 / BM))
    ea = jax.lax.broadcasted_iota(jnp.int32, (N_EXP, N_EXP), 0)
    eb = jax.lax.broadcasted_iota(jnp.int32, (N_EXP, N_EXP), 1)
    t8 = (ea < eb).astype(jnp.bfloat16)
    tile_base = jax.lax.dot_general(
        ntiles.astype(jnp.bfloat16), t8, (((1,), (0,)), ((), ())),
        preferred_element_type=jnp.float32)
    tb1 = jnp.sum(oh1 * tile_base, axis=-1, keepdims=True)
    tb2 = jnp.sum(oh2 * tile_base, axis=-1, keepdims=True)
    slot1 = (tb1 * BM + rank1).astype(jnp.int32)
    slot2 = (tb2 * BM + rank2).astype(jnp.int32)
    slot_ref[...] = jnp.concatenate([slot1, slot2], axis=1)

    tb_incl = tile_base + ntiles
    ji = jax.lax.broadcasted_iota(jnp.int32, (NT_PAD, N_EXP), 0).astype(jnp.float32)
    etile = jnp.sum((jnp.broadcast_to(tb_incl, (NT_PAD, N_EXP)) <= ji)
                    .astype(jnp.float32), axis=-1, keepdims=True)
    etile_ref[...] = jnp.minimum(etile, N_EXP - 1).astype(jnp.int32)


def _gather_kernel(tok_ref, x_ref, xs_ref):
    j = pl.program_id(0)

    def body(i, _):
        tok = tok_ref[j * BM + i]
        xs_ref[pl.ds(i, 1), :] = x_ref[pl.ds(tok, 1), :]
        return 0

    jax.lax.fori_loop(0, BM, body, 0)


def _ffn_kernel(e_ref, xs_ref, g_ref, we_ref, wv_ref, wo_ref,
                out_ref, rows_ref):
    n = pl.program_id(0)
    j = pl.program_id(1)
    xs = xs_ref[...]
    g = jax.lax.dot_general(xs, we_ref[0], (((1,), (1,)), ((), ())),
                            preferred_element_type=jnp.float32)
    v = jax.lax.dot_general(xs, wv_ref[0], (((1,), (1,)), ((), ())),
                            preferred_element_type=jnp.float32)
    h = (0.5 * g * (1.0 + jax.lax.erf(g * 0.7071067811865476))) * v
    h = h * g_ref[0]
    part = jax.lax.dot_general(h, wo_ref[0], (((1,), (1,)), ((), ())),
                               preferred_element_type=jnp.float32)
    base = j * BM

    @pl.when(n == 0)
    def _first():
        rows_ref[pl.ds(base, BM), :] = part

    @pl.when((n > 0) & (n < NN - 1))
    def _rest():
        rows_ref[pl.ds(base, BM), :] += part

    @pl.when(n == NN - 1)
    def _last():
        out_ref[...] = rows_ref[pl.ds(base, BM), :] + part


@jax.jit
def kernel(x, Wg, We, Wv, Wo):
    x2 = x.reshape(SEQ, D_MODEL)
    probs, gates2, slot2, etile = pl.pallas_call(
        _router_kernel,
        out_shape=(
            jax.ShapeDtypeStruct((SEQ, N_EXP), jnp.float32),
            jax.ShapeDtypeStruct((SEQ, K_SEL), jnp.float32),
            jax.ShapeDtypeStruct((SEQ, K_SEL), jnp.int32),
            jax.ShapeDtypeStruct((NT_PAD, 1), jnp.int32),
        ),
    )(x2, Wg)

    slot_flat = slot2.reshape(-1)
    tok_ids = jnp.arange(R_TOT, dtype=jnp.int32) // K_SEL
    tok_padded = jnp.zeros(NT * BM, jnp.int32).at[slot_flat].set(tok_ids)
    g_padded = (jnp.zeros(NT * BM, jnp.float32)
                .at[slot_flat].set(gates2.reshape(-1))
                .reshape(NT, BM, 1))
    e_of_tile = etile.reshape(NT_PAD)

    xs = x2[tok_padded]

    rows = pl.pallas_call(
        _ffn_kernel,
        grid_spec=pltpu.PrefetchScalarGridSpec(
            num_scalar_prefetch=1,
            grid=(NN, NT),
            in_specs=[
                pl.BlockSpec((BM, D_MODEL), lambda n, jt, e_t: (jt, 0)),
                pl.BlockSpec((1, BM, 1), lambda n, jt, e_t: (jt, 0, 0)),
                pl.BlockSpec((1, BN, D_MODEL),
                             lambda n, jt, e_t: (e_t[jt], n, 0)),
                pl.BlockSpec((1, BN, D_MODEL),
                             lambda n, jt, e_t: (e_t[jt], n, 0)),
                pl.BlockSpec((1, D_MODEL, BN),
                             lambda n, jt, e_t: (e_t[jt], 0, n)),
            ],
            out_specs=pl.BlockSpec(
                (BM, D_MODEL),
                lambda n, jt, e_t: (jnp.where(n == NN - 1, jt, NT), 0)),
            scratch_shapes=[pltpu.VMEM((NT * BM, D_MODEL), jnp.float32)],
        ),
        out_shape=jax.ShapeDtypeStruct(((NT + 1) * BM, D_MODEL), jnp.float32),
    )(e_of_tile, xs, g_padded, We, Wv, Wo)

    out = rows[slot2[:, 0]] + rows[slot2[:, 1]]
    return out.reshape(1, SEQ, D_MODEL), probs.reshape(1, SEQ, N_EXP)


# speedup vs baseline: 1.4332x; 1.1862x over previous
import jax
import jax.numpy as jnp
from jax.experimental import pallas as pl
from jax.experimental.pallas import tpu as pltpu

D_MODEL = 1024
D_FF = 4096
N_EXP = 8
SEQ = 2048
K_SEL = 2
R_TOT = SEQ * K_SEL
BM = 256
BN = 1024
NT = R_TOT // BM + N_EXP
NT_PAD = 32
NN = D_FF // BN


def _router_kernel(x_ref, wg_ref, probs_ref, gates_ref, slot_ref, etile_ref):
    x = x_ref[...]
    logits = jax.lax.dot_general(
        x, wg_ref[...], (((1,), (1,)), ((), ())),
        preferred_element_type=jnp.float32)
    m = jnp.max(logits, axis=-1, keepdims=True)
    ex = jnp.exp(logits - m)
    probs = ex / jnp.sum(ex, axis=-1, keepdims=True)
    probs_ref[...] = probs
    idx = jax.lax.broadcasted_iota(jnp.int32, probs.shape, 1)
    m1 = jnp.max(probs, axis=-1, keepdims=True)
    i1 = jnp.min(jnp.where(probs == m1, idx, N_EXP), axis=-1, keepdims=True)
    oh1 = (idx == i1).astype(jnp.float32)
    p2 = jnp.where(idx == i1, -1.0, probs)
    m2 = jnp.max(p2, axis=-1, keepdims=True)
    i2 = jnp.min(jnp.where(p2 == m2, idx, N_EXP), axis=-1, keepdims=True)
    oh2 = (idx == i2).astype(jnp.float32)
    denom = m1 + m2
    gates_ref[...] = jnp.concatenate([m1 / denom, m2 / denom], axis=1)

    both = (oh1 + oh2).astype(jnp.bfloat16)
    ia = jax.lax.broadcasted_iota(jnp.int32, (SEQ, SEQ), 0)
    ib = jax.lax.broadcasted_iota(jnp.int32, (SEQ, SEQ), 1)
    ltri = (ib < ia).astype(jnp.bfloat16)
    cum = jax.lax.dot_general(ltri, both, (((1,), (0,)), ((), ())),
                              preferred_element_type=jnp.float32)
    rank1 = jnp.sum(oh1 * cum, axis=-1, keepdims=True)
    rank2 = jnp.sum(oh2 * cum, axis=-1, keepdims=True)

    counts = jnp.sum(oh1 + oh2, axis=0, keepdims=True)
    ntiles = jnp.floor((counts + (BM - 1)) * (1.0 / BM))
    ea = jax.lax.broadcasted_iota(jnp.int32, (N_EXP, N_EXP), 0)
    eb = jax.lax.broadcasted_iota(jnp.int32, (N_EXP, N_EXP), 1)
    t8 = (ea < eb).astype(jnp.bfloat16)
    tile_base = jax.lax.dot_general(
        ntiles.astype(jnp.bfloat16), t8, (((1,), (0,)), ((), ())),
        preferred_element_type=jnp.float32)
    tb1 = jnp.sum(oh1 * tile_base, axis=-1, keepdims=True)
    tb2 = jnp.sum(oh2 * tile_base, axis=-1, keepdims=True)
    slot1 = (tb1 * BM + rank1).astype(jnp.int32)
    slot2 = (tb2 * BM + rank2).astype(jnp.int32)
    slot_ref[...] = jnp.concatenate([slot1, slot2], axis=1)

    tb_incl = tile_base + ntiles
    ji = jax.lax.broadcasted_iota(jnp.int32, (NT_PAD, N_EXP), 0).astype(jnp.float32)
    etile = jnp.sum((jnp.broadcast_to(tb_incl, (NT_PAD, N_EXP)) <= ji)
                    .astype(jnp.float32), axis=-1, keepdims=True)
    etile_ref[...] = jnp.minimum(etile, N_EXP - 1).astype(jnp.int32)


def _gather_kernel(tok_ref, x_ref, xs_ref):
    j = pl.program_id(0)

    def body(i, _):
        tok = tok_ref[j * BM + i]
        xs_ref[pl.ds(i, 1), :] = x_ref[pl.ds(tok, 1), :]
        return 0

    jax.lax.fori_loop(0, BM, body, 0)


def _ffn_kernel(e_ref, xs_ref, g_ref, we_ref, wv_ref, wo_ref,
                out_ref, rows_ref):
    n = pl.program_id(0)
    j = pl.program_id(1)
    xs = xs_ref[...]
    g = jax.lax.dot_general(xs, we_ref[0], (((1,), (1,)), ((), ())),
                            preferred_element_type=jnp.float32)
    v = jax.lax.dot_general(xs, wv_ref[0], (((1,), (1,)), ((), ())),
                            preferred_element_type=jnp.float32)
    h = (0.5 * g * (1.0 + jax.lax.erf(g * 0.7071067811865476))) * v
    h = h * g_ref[0]
    part = jax.lax.dot_general(h, wo_ref[0], (((1,), (1,)), ((), ())),
                               preferred_element_type=jnp.float32)
    base = j * BM

    @pl.when(n == 0)
    def _first():
        rows_ref[pl.ds(base, BM), :] = part

    @pl.when((n > 0) & (n < NN - 1))
    def _rest():
        rows_ref[pl.ds(base, BM), :] += part

    @pl.when(n == NN - 1)
    def _last():
        out_ref[...] = rows_ref[pl.ds(base, BM), :] + part


@jax.jit
def kernel(x, Wg, We, Wv, Wo):
    x2 = x.reshape(SEQ, D_MODEL)
    probs, gates2, slot2, etile = pl.pallas_call(
        _router_kernel,
        out_shape=(
            jax.ShapeDtypeStruct((SEQ, N_EXP), jnp.float32),
            jax.ShapeDtypeStruct((SEQ, K_SEL), jnp.float32),
            jax.ShapeDtypeStruct((SEQ, K_SEL), jnp.int32),
            jax.ShapeDtypeStruct((NT_PAD, 1), jnp.int32),
        ),
    )(x2, Wg)

    slot_flat = slot2.reshape(-1)
    tok_ids = jnp.arange(R_TOT, dtype=jnp.int32) // K_SEL
    tok_padded = jnp.zeros(NT * BM, jnp.int32).at[slot_flat].set(tok_ids)
    g_padded = (jnp.zeros(NT * BM, jnp.float32)
                .at[slot_flat].set(gates2.reshape(-1))
                .reshape(NT, BM, 1))
    e_of_tile = etile.reshape(NT_PAD)

    xs = x2[tok_padded]

    rows = pl.pallas_call(
        _ffn_kernel,
        grid_spec=pltpu.PrefetchScalarGridSpec(
            num_scalar_prefetch=1,
            grid=(NN, NT),
            in_specs=[
                pl.BlockSpec((BM, D_MODEL), lambda n, jt, e_t: (jt, 0)),
                pl.BlockSpec((1, BM, 1), lambda n, jt, e_t: (jt, 0, 0)),
                pl.BlockSpec((1, BN, D_MODEL),
                             lambda n, jt, e_t: (e_t[jt], n, 0)),
                pl.BlockSpec((1, BN, D_MODEL),
                             lambda n, jt, e_t: (e_t[jt], n, 0)),
                pl.BlockSpec((1, D_MODEL, BN),
                             lambda n, jt, e_t: (e_t[jt], 0, n)),
            ],
            out_specs=pl.BlockSpec(
                (BM, D_MODEL),
                lambda n, jt, e_t: (jnp.where(n == NN - 1, jt, NT), 0)),
            scratch_shapes=[pltpu.VMEM((NT * BM, D_MODEL), jnp.float32)],
        ),
        out_shape=jax.ShapeDtypeStruct(((NT + 1) * BM, D_MODEL), jnp.float32),
    )(e_of_tile, xs, g_padded, We, Wv, Wo)

    out = rows[slot2[:, 0]] + rows[slot2[:, 1]]
    return out.reshape(1, SEQ, D_MODEL), probs.reshape(1, SEQ, N_EXP)


# speedup vs baseline: 1.4581x; 1.0173x over previous
import jax
import jax.numpy as jnp
from jax.experimental import pallas as pl
from jax.experimental.pallas import tpu as pltpu

D_MODEL = 1024
D_FF = 4096
N_EXP = 8
SEQ = 2048
K_SEL = 2
R_TOT = SEQ * K_SEL
BM = 512
BN = 1024
NT = R_TOT // BM + N_EXP
NT_PAD = 32
NN = D_FF // BN


def _router_kernel(x_ref, wg_ref, probs_ref, gates_ref, slot_ref, etile_ref):
    x = x_ref[...]
    logits = jax.lax.dot_general(
        x, wg_ref[...], (((1,), (1,)), ((), ())),
        preferred_element_type=jnp.float32)
    m = jnp.max(logits, axis=-1, keepdims=True)
    ex = jnp.exp(logits - m)
    probs = ex / jnp.sum(ex, axis=-1, keepdims=True)
    probs_ref[...] = probs
    idx = jax.lax.broadcasted_iota(jnp.int32, probs.shape, 1)
    m1 = jnp.max(probs, axis=-1, keepdims=True)
    i1 = jnp.min(jnp.where(probs == m1, idx, N_EXP), axis=-1, keepdims=True)
    oh1 = (idx == i1).astype(jnp.float32)
    p2 = jnp.where(idx == i1, -1.0, probs)
    m2 = jnp.max(p2, axis=-1, keepdims=True)
    i2 = jnp.min(jnp.where(p2 == m2, idx, N_EXP), axis=-1, keepdims=True)
    oh2 = (idx == i2).astype(jnp.float32)
    denom = m1 + m2
    gates_ref[...] = jnp.concatenate([m1 / denom, m2 / denom], axis=1)

    both = (oh1 + oh2).astype(jnp.bfloat16)
    ia = jax.lax.broadcasted_iota(jnp.int32, (SEQ, SEQ), 0)
    ib = jax.lax.broadcasted_iota(jnp.int32, (SEQ, SEQ), 1)
    ltri = (ib < ia).astype(jnp.bfloat16)
    cum = jax.lax.dot_general(ltri, both, (((1,), (0,)), ((), ())),
                              preferred_element_type=jnp.float32)
    rank1 = jnp.sum(oh1 * cum, axis=-1, keepdims=True)
    rank2 = jnp.sum(oh2 * cum, axis=-1, keepdims=True)

    counts = jnp.sum(oh1 + oh2, axis=0, keepdims=True)
    ntiles = jnp.floor((counts + (BM - 1)) * (1.0 / BM))
    ea = jax.lax.broadcasted_iota(jnp.int32, (N_EXP, N_EXP), 0)
    eb = jax.lax.broadcasted_iota(jnp.int32, (N_EXP, N_EXP), 1)
    t8 = (ea < eb).astype(jnp.bfloat16)
    tile_base = jax.lax.dot_general(
        ntiles.astype(jnp.bfloat16), t8, (((1,), (0,)), ((), ())),
        preferred_element_type=jnp.float32)
    tb1 = jnp.sum(oh1 * tile_base, axis=-1, keepdims=True)
    tb2 = jnp.sum(oh2 * tile_base, axis=-1, keepdims=True)
    slot1 = (tb1 * BM + rank1).astype(jnp.int32)
    slot2 = (tb2 * BM + rank2).astype(jnp.int32)
    slot_ref[...] = jnp.concatenate([slot1, slot2], axis=1)

    tb_incl = tile_base + ntiles
    ji = jax.lax.broadcasted_iota(jnp.int32, (NT_PAD, N_EXP), 0).astype(jnp.float32)
    etile = jnp.sum((jnp.broadcast_to(tb_incl, (NT_PAD, N_EXP)) <= ji)
                    .astype(jnp.float32), axis=-1, keepdims=True)
    etile_ref[...] = jnp.minimum(etile, N_EXP - 1).astype(jnp.int32)


def _gather_kernel(tok_ref, x_ref, xs_ref):
    j = pl.program_id(0)

    def body(i, _):
        tok = tok_ref[j * BM + i]
        xs_ref[pl.ds(i, 1), :] = x_ref[pl.ds(tok, 1), :]
        return 0

    jax.lax.fori_loop(0, BM, body, 0)


def _ffn_kernel(e_ref, xs_ref, g_ref, we_ref, wv_ref, wo_ref,
                out_ref, rows_ref):
    n = pl.program_id(0)
    j = pl.program_id(1)
    xs = xs_ref[...]
    g = jax.lax.dot_general(xs, we_ref[0], (((1,), (1,)), ((), ())),
                            preferred_element_type=jnp.float32)
    v = jax.lax.dot_general(xs, wv_ref[0], (((1,), (1,)), ((), ())),
                            preferred_element_type=jnp.float32)
    h = (0.5 * g * (1.0 + jax.lax.erf(g * 0.7071067811865476))) * v
    h = h * g_ref[0]
    part = jax.lax.dot_general(h, wo_ref[0], (((1,), (1,)), ((), ())),
                               preferred_element_type=jnp.float32)
    base = j * BM

    @pl.when(n == 0)
    def _first():
        rows_ref[pl.ds(base, BM), :] = part.astype(jnp.bfloat16)

    @pl.when((n > 0) & (n < NN - 1))
    def _rest():
        rows_ref[pl.ds(base, BM), :] = (
            rows_ref[pl.ds(base, BM), :].astype(jnp.float32) + part
        ).astype(jnp.bfloat16)

    @pl.when(n == NN - 1)
    def _last():
        out_ref[...] = rows_ref[pl.ds(base, BM), :].astype(jnp.float32) + part


@jax.jit
def kernel(x, Wg, We, Wv, Wo):
    x2 = x.reshape(SEQ, D_MODEL)
    probs, gates2, slot2, etile = pl.pallas_call(
        _router_kernel,
        out_shape=(
            jax.ShapeDtypeStruct((SEQ, N_EXP), jnp.float32),
            jax.ShapeDtypeStruct((SEQ, K_SEL), jnp.float32),
            jax.ShapeDtypeStruct((SEQ, K_SEL), jnp.int32),
            jax.ShapeDtypeStruct((NT_PAD, 1), jnp.int32),
        ),
    )(x2, Wg)

    slot_flat = slot2.reshape(-1)
    tok_ids = jnp.arange(R_TOT, dtype=jnp.int32) // K_SEL
    tok_padded = jnp.zeros(NT * BM, jnp.int32).at[slot_flat].set(tok_ids)
    g_padded = (jnp.zeros(NT * BM, jnp.float32)
                .at[slot_flat].set(gates2.reshape(-1))
                .reshape(NT, BM, 1))
    e_of_tile = etile.reshape(NT_PAD)

    xs = x2[tok_padded]

    rows = pl.pallas_call(
        _ffn_kernel,
        grid_spec=pltpu.PrefetchScalarGridSpec(
            num_scalar_prefetch=1,
            grid=(NN, NT),
            in_specs=[
                pl.BlockSpec((BM, D_MODEL), lambda n, jt, e_t: (jt, 0)),
                pl.BlockSpec((1, BM, 1), lambda n, jt, e_t: (jt, 0, 0)),
                pl.BlockSpec((1, BN, D_MODEL),
                             lambda n, jt, e_t: (e_t[jt], n, 0)),
                pl.BlockSpec((1, BN, D_MODEL),
                             lambda n, jt, e_t: (e_t[jt], n, 0)),
                pl.BlockSpec((1, D_MODEL, BN),
                             lambda n, jt, e_t: (e_t[jt], 0, n)),
            ],
            out_specs=pl.BlockSpec(
                (BM, D_MODEL),
                lambda n, jt, e_t: (jnp.where(n == NN - 1, jt, NT), 0)),
            scratch_shapes=[pltpu.VMEM((NT * BM, D_MODEL), jnp.bfloat16)],
        ),
        out_shape=jax.ShapeDtypeStruct(((NT + 1) * BM, D_MODEL), jnp.float32),
    )(e_of_tile, xs, g_padded, We, Wv, Wo)

    out = rows[slot2[:, 0]] + rows[slot2[:, 1]]
    return out.reshape(1, SEQ, D_MODEL), probs.reshape(1, SEQ, N_EXP)
